# baseline (speedup 1.0000x reference)
import jax
import jax.numpy as jnp
from jax.experimental import pallas as pl
from jax.experimental.pallas import tpu as pltpu

_IN = 32
_HID = 64
_OUT = 16
_FUSED = _HID + _OUT


def _spike_body(x_ref, w13_ref, b13_ref, w2n_ref, b2n_ref,
                spk1_ref, spk2_ref, spk_out_ref):
    h13 = jnp.dot(x_ref[...], w13_ref[...],
                  preferred_element_type=jnp.float32) + b13_ref[...]
    h1t = h13[:, :_HID]
    h3 = h13[:, _HID:]

    spk1 = jnp.where(h1t > 0.0, 1.0, 0.0)
    spk1_ref[...] = spk1

    h2t = jnp.dot(spk1, w2n_ref[...],
                  preferred_element_type=jnp.float32) + b2n_ref[...]
    spk2_ref[...] = jnp.where(h2t > 0.0, 1.0, 0.0)
    spk_out_ref[...] = jnp.where(h2t + h3 > 0.0, 1.0, 0.0)


def _row_tile(n):
    for t in (4096, 2048, 1024, 512, 256, 128, 64, 32, 16, 8):
        if n % t == 0 and n // t >= 2:
            return t
    return n


def kernel(x, w13, b13, w2n, b2n):
    n = x.shape[0]
    tile = _row_tile(n)
    grid = (n // tile,)

    vmem = pltpu.MemorySpace.VMEM
    full = lambda i: (0, 0)
    rows = lambda i: (i, 0)

    flops = 2 * n * _IN * _FUSED + 2 * n * _HID * _OUT + 4 * n * _OUT
    nbytes = 4 * (n * (_IN + _HID + 2 * _OUT)
                  + w13.size + b13.size + w2n.size + b2n.size)

    return pl.pallas_call(
        _spike_body,
        grid=grid,
        in_specs=[
            pl.BlockSpec((tile, _IN), rows, memory_space=vmem),
            pl.BlockSpec((_IN, _FUSED), full, memory_space=vmem),
            pl.BlockSpec((1, _FUSED), full, memory_space=vmem),
            pl.BlockSpec((_HID, _OUT), full, memory_space=vmem),
            pl.BlockSpec((1, _OUT), full, memory_space=vmem),
        ],
        out_specs=(
            pl.BlockSpec((tile, _HID), rows, memory_space=vmem),
            pl.BlockSpec((tile, _OUT), rows, memory_space=vmem),
            pl.BlockSpec((tile, _OUT), rows, memory_space=vmem),
        ),
        out_shape=(
            jax.ShapeDtypeStruct((n, _HID), jnp.float32),
            jax.ShapeDtypeStruct((n, _OUT), jnp.float32),
            jax.ShapeDtypeStruct((n, _OUT), jnp.float32),
        ),
        compiler_params=pltpu.CompilerParams(
            dimension_semantics=("parallel",)),
        cost_estimate=pl.CostEstimate(flops=flops, transcendentals=0,
                                      bytes_accessed=nbytes),
    )(x, w13, b13, w2n, b2n)


# speedup vs baseline: 1.0127x; 1.0127x over previous
import jax
import jax.numpy as jnp
from jax.experimental import pallas as pl
from jax.experimental.pallas import tpu as pltpu

_IN = 32
_HID = 64
_OUT = 16
_FUSED = _HID + _OUT


def _spike_body(x_ref, w13_ref, b13_ref, w2n_ref, b2n_ref,
                spk1_ref, spk2_ref, spk_out_ref):
    h13 = jnp.dot(x_ref[...], w13_ref[...],
                  preferred_element_type=jnp.float32) + b13_ref[...]
    h1t = h13[:, :_HID]
    h3 = h13[:, _HID:]

    spk1 = jnp.where(h1t > 0.0, 1.0, 0.0)
    spk1_ref[...] = spk1

    h2t = jnp.dot(spk1, w2n_ref[...],
                  preferred_element_type=jnp.float32) + b2n_ref[...]
    spk2_ref[...] = jnp.where(h2t > 0.0, 1.0, 0.0)
    spk_out_ref[...] = jnp.where(h2t + h3 > 0.0, 1.0, 0.0)


def _row_tile(n):
    for t in (8192, 4096, 2048, 1024, 512, 256, 128, 64, 32, 16, 8):
        if n % t == 0 and n // t >= 2:
            return t
    return n


def kernel(x, w13, b13, w2n, b2n):
    n = x.shape[0]
    tile = _row_tile(n)
    grid = (n // tile,)

    vmem = pltpu.MemorySpace.VMEM
    full = lambda i: (0, 0)
    rows = lambda i: (i, 0)

    flops = 2 * n * _IN * _FUSED + 2 * n * _HID * _OUT + 4 * n * _OUT
    nbytes = 4 * (n * (_IN + _HID + 2 * _OUT)
                  + w13.size + b13.size + w2n.size + b2n.size)

    return pl.pallas_call(
        _spike_body,
        grid=grid,
        in_specs=[
            pl.BlockSpec((tile, _IN), rows, memory_space=vmem),
            pl.BlockSpec((_IN, _FUSED), full, memory_space=vmem),
            pl.BlockSpec((1, _FUSED), full, memory_space=vmem),
            pl.BlockSpec((_HID, _OUT), full, memory_space=vmem),
            pl.BlockSpec((1, _OUT), full, memory_space=vmem),
        ],
        out_specs=(
            pl.BlockSpec((tile, _HID), rows, memory_space=vmem),
            pl.BlockSpec((tile, _OUT), rows, memory_space=vmem),
            pl.BlockSpec((tile, _OUT), rows, memory_space=vmem),
        ),
        out_shape=(
            jax.ShapeDtypeStruct((n, _HID), jnp.float32),
            jax.ShapeDtypeStruct((n, _OUT), jnp.float32),
            jax.ShapeDtypeStruct((n, _OUT), jnp.float32),
        ),
        compiler_params=pltpu.CompilerParams(
            dimension_semantics=("parallel",)),
        cost_estimate=pl.CostEstimate(flops=flops, transcendentals=0,
                                      bytes_accessed=nbytes),
    )(x, w13, b13, w2n, b2n)
